# baseline (speedup 1.0000x reference)
import jax
import jax.numpy as jnp
from jax.experimental import pallas as pl
from jax.experimental.pallas import tpu as pltpu

_C_IN = 7
_C_OUT = 4
_K = 5
_CP = 8
_H = 64
_W = 64
_HP = _H + 2 * (_K - 1)
_WP = _W + 2 * (_K - 1)
_HO = _H + _K - 1
_WO = _W + _K - 1
_L_OUT = _HO * _WP
_SEG = 4992
_L_IN = 5376
_KR = _K * _CP
_NB = 4


def _body(w_ref, b_ref, x_ref, o_ref, xp_ref, xs_ref, p_ref):
    for nb in range(_NB):
        xcol = nb * _L_IN
        xp_ref[:, pl.ds(xcol, _L_IN)] = jnp.zeros((_CP, _L_IN), jnp.bfloat16)
        for h in range(_H):
            dst = xcol + (h + _K - 1) * _WP + (_K - 1)
            xp_ref[0:_C_IN, pl.ds(dst, _W)] = x_ref[
                nb, :, pl.ds(h * _W, _W)].astype(jnp.bfloat16)

        col = nb * _SEG
        for kh in range(_K):
            xs_ref[pl.ds(kh * _CP, _CP), pl.ds(col, _SEG)] = (
                xp_ref[:, pl.ds(xcol + kh * _WP, _SEG)])

        p_ref[:, pl.ds(col, _SEG)] = jnp.dot(
            w_ref[...], xs_ref[:, pl.ds(col, _SEG)],
            preferred_element_type=jnp.float32)

    for nb in range(_NB):
        col = nb * _SEG
        v = p_ref[0:_CP, pl.ds(col, _L_OUT)] + b_ref[:, 0:1]
        for kw in range(1, _K):
            v = v + p_ref[pl.ds(kw * _CP, _CP), pl.ds(col + kw, _L_OUT)]

        inner = v * (0.7978845608028654 + 0.035677408136300125 * (v * v))
        g = 0.5 * v * (jnp.tanh(inner) + 1.0)
        o_ref[nb] = g[:_C_OUT].astype(o_ref.dtype)


def _build_weight_mat(weight, bias):
    w_flip = weight[:, :, ::-1, ::-1]
    w_flip = jnp.pad(
        w_flip, ((0, _CP - _C_IN), (0, _CP - _C_OUT), (0, 0), (0, 0)))
    arr = jnp.transpose(w_flip, (3, 1, 2, 0))
    w_mat = arr.reshape(_KR, _KR)
    b_col = jnp.pad(bias, (0, _CP - _C_OUT))
    b_mat = jnp.broadcast_to(b_col[:, None], (_CP, 128)).astype(jnp.float32)
    return w_mat.astype(jnp.bfloat16), b_mat


@jax.jit
def _run(x_nchw, weight, bias):
    n = x_nchw.shape[0]
    x_flat = x_nchw.reshape(n, _C_IN, _H * _W)
    w_mat, b_mat = _build_weight_mat(weight, bias)

    out = pl.pallas_call(
        _body,
        out_shape=jax.ShapeDtypeStruct((n, _C_OUT, _L_OUT), jnp.float32),
        grid=(n // _NB,),
        in_specs=[
            pl.BlockSpec((_KR, _KR), lambda i: (0, 0)),
            pl.BlockSpec((_CP, 128), lambda i: (0, 0)),
            pl.BlockSpec((_NB, _C_IN, _H * _W), lambda i: (i, 0, 0)),
        ],
        out_specs=pl.BlockSpec((_NB, _C_OUT, _L_OUT), lambda i: (i, 0, 0)),
        scratch_shapes=[
            pltpu.VMEM((_CP, _NB * _L_IN), jnp.bfloat16),
            pltpu.VMEM((_KR, _NB * _SEG), jnp.bfloat16),
            pltpu.VMEM((_KR, _NB * _SEG), jnp.float32),
        ],
        compiler_params=pltpu.CompilerParams(
            dimension_semantics=("parallel",)),
    )(w_mat, b_mat, x_flat)

    y = out.reshape(n, _C_OUT, _HO, _WP)
    return y[:, :, :, :_WO]


def kernel(x_nchw, weight, bias):
    return _run(x_nchw, weight, bias)
